# baseline (speedup 1.0000x reference)
import functools

import jax
import jax.numpy as jnp
from jax import lax
from jax.experimental import pallas as pl
from jax.experimental.pallas import tpu as pltpu
from jax.experimental.pallas import tpu_sc as plsc

N_NODES = 10000
N_PAD = 10240
N_EDGES = 320000
D = 128
NC = 2
NS = 16
NW = NC * NS
E_PER_W = N_EDGES // NW
K = 125
NCHUNK = E_PER_W // K
NBUF = 2
ROWS_PER_TILE = N_PAD // NS
ZR = 80

_mesh = plsc.VectorSubcoreMesh(core_axis_name="c", subcore_axis_name="s")


def _vec_rsqrt(d):
    magic = jnp.full((16,), 0x5F3759DF, jnp.int32)
    one = jnp.full((16,), 1, jnp.int32)
    c15 = jnp.full((16,), 1.5, jnp.float32)
    ch = jnp.full((16,), 0.5, jnp.float32)
    i = lax.bitcast_convert_type(d, jnp.int32)
    i = magic - lax.shift_right_arithmetic(i, one)
    y = lax.bitcast_convert_type(i, jnp.float32)
    for _ in range(3):
        y = y * (c15 - ch * d * y * y)
    return y


@functools.partial(
    pl.kernel,
    out_type=(
        jax.ShapeDtypeStruct((NC, N_PAD, D), jnp.float32),
        jax.ShapeDtypeStruct((N_PAD, D), jnp.float32),
        jax.ShapeDtypeStruct((N_PAD,), jnp.float32),
    ),
    mesh=_mesh,
    scratch_types=[
        pltpu.VMEM((NCHUNK, K), jnp.int32),
        [pltpu.VMEM((K,), jnp.int32)] * NBUF,
        [pltpu.VMEM((K, D), jnp.float32)] * NBUF,
        pltpu.VMEM((K,), jnp.float32),
        pltpu.VMEM((ROWS_PER_TILE,), jnp.float32),
        pltpu.VMEM((ROWS_PER_TILE,), jnp.float32),
        pltpu.VMEM_SHARED((N_PAD, D), jnp.float32),
        pltpu.VMEM_SHARED((N_PAD,), jnp.float32),
        pltpu.SemaphoreType.DMA,
        pltpu.SemaphoreType.DMA,
        pltpu.SemaphoreType.DMA,
    ],
)
def _fused_kernel(x_hbm, col_hbm, row_hbm, acc_hbm, u_hbm, deg_hbm,
                  col_v, rowb, bufs, ones_v, zv, dinvv,
                  acc_sh, bins_sh, gsem, rsem, hsem):
    cid = lax.axis_index("c")
    sid = lax.axis_index("s")
    wid = sid * NC + cid
    r0 = sid * ROWS_PER_TILE
    tile_sl = pl.ds(r0, ROWS_PER_TILE)

    def zbody(i, _):
        r = i // (D // 16)
        c = lax.rem(i, D // 16)
        bufs[0][r, pl.ds(c * 16, 16)] = jnp.zeros((16,), jnp.float32)
        return 0

    lax.fori_loop(0, ZR * (D // 16), zbody, 0)

    def z1body(i, _):
        zv[pl.ds(i * 16, 16)] = jnp.zeros((16,), jnp.float32)
        return 0

    lax.fori_loop(0, ROWS_PER_TILE // 16, z1body, 0)
    for i in range(K // 16 + 1):
        o = min(i * 16, K - 16)
        ones_v[pl.ds(o, 16)] = jnp.ones((16,), jnp.float32)

    pltpu.sync_copy(zv, bins_sh.at[tile_sl])

    @pl.when(cid == 1)
    def _():
        zsrc = bufs[0].at[pl.ds(0, ZR), :]
        for i in range(ROWS_PER_TILE // ZR):
            pltpu.sync_copy(zsrc, acc_sh.at[pl.ds(r0 + i * ZR, ZR), :])

    plsc.subcore_barrier()

    GRP = 8
    for w01 in range(2):
        pltpu.sync_copy(row_hbm.at[sid * 2 + w01], col_v)

        def hbody(j, _):
            descs = []
            for b in range(GRP):
                descs.append(pltpu.async_copy(
                    ones_v, bins_sh.at[col_v.at[j * GRP + b]], hsem,
                    add=True))
            for dsc in descs:
                dsc.wait()
            return 0

        lax.fori_loop(0, NCHUNK // GRP, hbody, 0)
    plsc.subcore_barrier()

    pltpu.sync_copy(bins_sh.at[tile_sl], zv)

    @pl.when(cid == 0)
    def _():
        pltpu.sync_copy(bins_sh.at[tile_sl], deg_hbm.at[tile_sl])

    def dbody(i, _):
        sl = pl.ds(i * 16, 16)
        dinvv[sl] = _vec_rsqrt(zv[sl] + 1.0)
        return 0

    lax.fori_loop(0, ROWS_PER_TILE // 16, dbody, 0)

    NUC = ROWS_PER_TILE // ZR
    for i in range(NBUF):
        pltpu.async_copy(
            x_hbm.at[pl.ds(r0 + i * ZR, ZR), :],
            bufs[i].at[pl.ds(0, ZR), :], gsem)

    def ubody(i2, _):
        for b in range(NBUF):
            i = i2 * NBUF + b
            xsl = pl.ds(r0 + i * ZR, ZR)
            pltpu.make_async_copy(
                x_hbm.at[xsl, :], bufs[b].at[pl.ds(0, ZR), :], gsem).wait()

            def sgroup(gidx, _):
                dv = dinvv[pl.ds(i * ZR + gidx * 16, 16)]
                for r in range(16):
                    s = dv[r]
                    row = gidx * 16 + r
                    for c in range(D // 16):
                        lsl = pl.ds(c * 16, 16)
                        bufs[b][row, lsl] = bufs[b][row, lsl] * s
                return 0

            lax.fori_loop(0, ZR // 16, sgroup, 0)
            pltpu.sync_copy(bufs[b].at[pl.ds(0, ZR), :], u_hbm.at[xsl, :])

            @pl.when(cid == 0)
            def _():
                pltpu.sync_copy(
                    bufs[b].at[pl.ds(0, ZR), :], acc_sh.at[xsl, :])

            @pl.when(i + NBUF < NUC)
            def _():
                pltpu.async_copy(
                    x_hbm.at[pl.ds(r0 + (i + NBUF) * ZR, ZR), :],
                    bufs[b].at[pl.ds(0, ZR), :], gsem)

        return 0

    lax.fori_loop(0, NUC // NBUF, ubody, 0)

    pltpu.sync_copy(col_hbm.at[wid], col_v)
    plsc.subcore_barrier()

    for b in range(NBUF):
        pltpu.async_copy(row_hbm.at[wid, b], rowb[b], rsem)
        pltpu.async_copy(u_hbm.at[col_v.at[b]], bufs[b], gsem)

    def body(j, _):
        for b in range(NBUF):
            jj = j * NBUF + b
            pltpu.make_async_copy(row_hbm.at[wid, jj], rowb[b], rsem).wait()
            pltpu.make_async_copy(u_hbm.at[col_v.at[jj]], bufs[b], gsem).wait()
            pltpu.sync_copy(bufs[b], acc_sh.at[rowb[b]], add=True)

            @pl.when(jj + NBUF < NCHUNK)
            def _():
                pltpu.async_copy(row_hbm.at[wid, jj + NBUF], rowb[b], rsem)
                pltpu.async_copy(u_hbm.at[col_v.at[jj + NBUF]], bufs[b], gsem)

        return 0

    lax.fori_loop(0, NCHUNK // NBUF, body, 0)
    plsc.subcore_barrier()
    pltpu.sync_copy(acc_sh.at[tile_sl, :], acc_hbm.at[cid, tile_sl, :])


BR = 512


@functools.partial(
    pl.pallas_call,
    out_shape=jax.ShapeDtypeStruct((N_PAD, D), jnp.float32),
    grid=(N_PAD // BR,),
    in_specs=[
        pl.BlockSpec((NC, BR, D), lambda i: (0, i, 0)),
        pl.BlockSpec((D, D), lambda i: (0, 0)),
        pl.BlockSpec((BR, 1), lambda i: (i, 0)),
        pl.BlockSpec((1, D), lambda i: (0, 0)),
    ],
    out_specs=pl.BlockSpec((BR, D), lambda i: (i, 0)),
)
def _final_matmul(acc_ref, w_ref, deg_ref, b_ref, out_ref):
    dinv = lax.rsqrt(deg_ref[...] + 1.0)
    s = acc_ref[0] + acc_ref[1]
    y = jnp.dot(s, w_ref[...], preferred_element_type=jnp.float32)
    out_ref[...] = dinv * y + b_ref[...]


def kernel(x, edge_index, W, b):
    row = edge_index[0].reshape(NW, NCHUNK, K)
    col = edge_index[1].reshape(NW, NCHUNK, K)
    x_pad = jnp.pad(x, ((0, N_PAD - N_NODES), (0, 0)))
    acc, _u, deg = _fused_kernel(x_pad, col, row)
    out = _final_matmul(acc, W, deg.reshape(N_PAD, 1), b.reshape(1, D))
    return out[:N_NODES]


# speedup vs baseline: 1.0789x; 1.0789x over previous
import functools

import jax
import jax.numpy as jnp
from jax import lax
from jax.experimental import pallas as pl
from jax.experimental.pallas import tpu as pltpu
from jax.experimental.pallas import tpu_sc as plsc

N_NODES = 10000
N_PAD = 10240
N_EDGES = 320000
D = 128
NC = 2
NS = 16
NW = NC * NS
E_PER_W = N_EDGES // NW
K = 125
NCHUNK = E_PER_W // K
NBUF = 2
ROWS_PER_TILE = N_PAD // NS
ZR = 80

_mesh = plsc.VectorSubcoreMesh(core_axis_name="c", subcore_axis_name="s")


def _vec_rsqrt(d):
    magic = jnp.full((16,), 0x5F3759DF, jnp.int32)
    one = jnp.full((16,), 1, jnp.int32)
    c15 = jnp.full((16,), 1.5, jnp.float32)
    ch = jnp.full((16,), 0.5, jnp.float32)
    i = lax.bitcast_convert_type(d, jnp.int32)
    i = magic - lax.shift_right_arithmetic(i, one)
    y = lax.bitcast_convert_type(i, jnp.float32)
    for _ in range(3):
        y = y * (c15 - ch * d * y * y)
    return y


@functools.partial(
    pl.kernel,
    out_type=(
        jax.ShapeDtypeStruct((NC, N_PAD, D), jnp.float32),
        jax.ShapeDtypeStruct((N_PAD, D), jnp.float32),
        jax.ShapeDtypeStruct((N_PAD,), jnp.float32),
    ),
    mesh=_mesh,
    scratch_types=[
        pltpu.VMEM((NCHUNK, K), jnp.int32),
        [pltpu.VMEM((K,), jnp.int32)] * NBUF,
        [pltpu.VMEM((K, D), jnp.float32)] * NBUF,
        pltpu.VMEM((K,), jnp.float32),
        pltpu.VMEM((ROWS_PER_TILE,), jnp.float32),
        pltpu.VMEM((ROWS_PER_TILE,), jnp.float32),
        pltpu.VMEM_SHARED((N_PAD, D), jnp.float32),
        pltpu.VMEM_SHARED((N_PAD,), jnp.float32),
        pltpu.SemaphoreType.DMA,
        pltpu.SemaphoreType.DMA,
        pltpu.SemaphoreType.DMA,
    ],
)
def _fused_kernel(x_hbm, col_hbm, row_hbm, acc_hbm, u_hbm, deg_hbm,
                  col_v, rowb, bufs, ones_v, zv, dinvv,
                  acc_sh, bins_sh, gsem, rsem, hsem):
    cid = lax.axis_index("c")
    sid = lax.axis_index("s")
    wid = sid * NC + cid
    r0 = sid * ROWS_PER_TILE
    tile_sl = pl.ds(r0, ROWS_PER_TILE)

    def zbody(i, _):
        r = i // (D // 16)
        c = lax.rem(i, D // 16)
        bufs[0][r, pl.ds(c * 16, 16)] = jnp.zeros((16,), jnp.float32)
        return 0

    lax.fori_loop(0, ZR * (D // 16), zbody, 0)

    def z1body(i, _):
        zv[pl.ds(i * 16, 16)] = jnp.zeros((16,), jnp.float32)
        return 0

    lax.fori_loop(0, ROWS_PER_TILE // 16, z1body, 0)
    for i in range(K // 16 + 1):
        o = min(i * 16, K - 16)
        ones_v[pl.ds(o, 16)] = jnp.ones((16,), jnp.float32)

    pltpu.sync_copy(zv, bins_sh.at[tile_sl])

    @pl.when(cid == 1)
    def _():
        zsrc = bufs[0].at[pl.ds(0, ZR), :]
        for i in range(ROWS_PER_TILE // ZR):
            pltpu.sync_copy(zsrc, acc_sh.at[pl.ds(r0 + i * ZR, ZR), :])

    plsc.subcore_barrier()

    GRP = 8
    for w01 in range(2):
        pltpu.sync_copy(row_hbm.at[sid * 2 + w01], col_v)

        def hbody(j, _):
            descs = []
            for b in range(GRP):
                descs.append(pltpu.async_copy(
                    ones_v, bins_sh.at[col_v.at[j * GRP + b]], hsem,
                    add=True))
            for dsc in descs:
                dsc.wait()
            return 0

        lax.fori_loop(0, NCHUNK // GRP, hbody, 0)
    plsc.subcore_barrier()

    pltpu.sync_copy(bins_sh.at[tile_sl], zv)

    @pl.when(cid == 0)
    def _():
        pltpu.sync_copy(bins_sh.at[tile_sl], deg_hbm.at[tile_sl])

    def dbody(i, _):
        sl = pl.ds(i * 16, 16)
        dinvv[sl] = _vec_rsqrt(zv[sl] + 1.0)
        return 0

    lax.fori_loop(0, ROWS_PER_TILE // 16, dbody, 0)

    NUC = ROWS_PER_TILE // ZR
    nuc = jnp.where(r0 + ROWS_PER_TILE > N_NODES, (N_NODES - r0) // ZR, NUC)
    for i in range(NBUF):
        pltpu.async_copy(
            x_hbm.at[pl.ds(r0 + i * ZR, ZR), :],
            bufs[i].at[pl.ds(0, ZR), :], gsem)

    def ubody(i2, _):
        for b in range(NBUF):
            i = i2 * NBUF + b

            @pl.when(i < nuc)
            def _():
                xsl = pl.ds(r0 + i * ZR, ZR)
                pltpu.make_async_copy(
                    x_hbm.at[xsl, :], bufs[b].at[pl.ds(0, ZR), :], gsem).wait()

                def sgroup(gidx, _):
                    dv = dinvv[pl.ds(i * ZR + gidx * 16, 16)]
                    for r in range(16):
                        s = dv[r]
                        row = gidx * 16 + r
                        for c in range(D // 16):
                            lsl = pl.ds(c * 16, 16)
                            bufs[b][row, lsl] = bufs[b][row, lsl] * s
                    return 0

                lax.fori_loop(0, ZR // 16, sgroup, 0)
                pltpu.sync_copy(bufs[b].at[pl.ds(0, ZR), :], u_hbm.at[xsl, :])

                @pl.when(cid == 0)
                def _():
                    pltpu.sync_copy(
                        bufs[b].at[pl.ds(0, ZR), :], acc_sh.at[xsl, :])

                @pl.when(i + NBUF < nuc)
                def _():
                    pltpu.async_copy(
                        x_hbm.at[pl.ds(r0 + (i + NBUF) * ZR, ZR), :],
                        bufs[b].at[pl.ds(0, ZR), :], gsem)

        return 0

    lax.fori_loop(0, NUC // NBUF, ubody, 0)

    pltpu.sync_copy(col_hbm.at[wid], col_v)
    plsc.subcore_barrier()

    for b in range(NBUF):
        pltpu.async_copy(row_hbm.at[wid, b], rowb[b], rsem)
        pltpu.async_copy(u_hbm.at[col_v.at[b]], bufs[b], gsem)

    def body(j, _):
        for b in range(NBUF):
            jj = j * NBUF + b
            pltpu.make_async_copy(row_hbm.at[wid, jj], rowb[b], rsem).wait()
            pltpu.make_async_copy(u_hbm.at[col_v.at[jj]], bufs[b], gsem).wait()
            pltpu.sync_copy(bufs[b], acc_sh.at[rowb[b]], add=True)

            @pl.when(jj + NBUF < NCHUNK)
            def _():
                pltpu.async_copy(row_hbm.at[wid, jj + NBUF], rowb[b], rsem)
                pltpu.async_copy(u_hbm.at[col_v.at[jj + NBUF]], bufs[b], gsem)

        return 0

    lax.fori_loop(0, NCHUNK // NBUF, body, 0)
    plsc.subcore_barrier()
    pltpu.sync_copy(acc_sh.at[tile_sl, :], acc_hbm.at[cid, tile_sl, :])


BR = 1000


@functools.partial(
    pl.pallas_call,
    out_shape=jax.ShapeDtypeStruct((N_NODES, D), jnp.float32),
    grid=(N_NODES // BR,),
    in_specs=[
        pl.BlockSpec((NC, BR, D), lambda i: (0, i, 0)),
        pl.BlockSpec((D, D), lambda i: (0, 0)),
        pl.BlockSpec((BR, 1), lambda i: (i, 0)),
        pl.BlockSpec((1, D), lambda i: (0, 0)),
    ],
    out_specs=pl.BlockSpec((BR, D), lambda i: (i, 0)),
)
def _final_matmul(acc_ref, w_ref, deg_ref, b_ref, out_ref):
    dinv = lax.rsqrt(deg_ref[...] + 1.0)
    s = acc_ref[0] + acc_ref[1]
    y = jnp.dot(s.astype(jnp.bfloat16), w_ref[...].astype(jnp.bfloat16),
                preferred_element_type=jnp.float32)
    out_ref[...] = dinv * y + b_ref[...]


def kernel(x, edge_index, W, b):
    row = edge_index[0].reshape(NW, NCHUNK, K)
    col = edge_index[1].reshape(NW, NCHUNK, K)
    acc, _u, deg = _fused_kernel(x, col, row)
    out = _final_matmul(acc, W, deg.reshape(N_PAD, 1), b.reshape(1, D))
    return out


# speedup vs baseline: 1.0893x; 1.0097x over previous
import functools

import jax
import jax.numpy as jnp
from jax import lax
from jax.experimental import pallas as pl
from jax.experimental.pallas import tpu as pltpu
from jax.experimental.pallas import tpu_sc as plsc

N_NODES = 10000
N_PAD = 10240
N_EDGES = 320000
D = 128
NC = 2
NS = 16
NW = NC * NS
E_PER_W = N_EDGES // NW
K = 125
NCHUNK = E_PER_W // K
NBUF = 2
ROWS_PER_TILE = N_PAD // NS
ZR = 80

_mesh = plsc.VectorSubcoreMesh(core_axis_name="c", subcore_axis_name="s")


def _vec_rsqrt(d):
    magic = jnp.full((16,), 0x5F3759DF, jnp.int32)
    one = jnp.full((16,), 1, jnp.int32)
    c15 = jnp.full((16,), 1.5, jnp.float32)
    ch = jnp.full((16,), 0.5, jnp.float32)
    i = lax.bitcast_convert_type(d, jnp.int32)
    i = magic - lax.shift_right_arithmetic(i, one)
    y = lax.bitcast_convert_type(i, jnp.float32)
    for _ in range(3):
        y = y * (c15 - ch * d * y * y)
    return y


@functools.partial(
    pl.kernel,
    out_type=(
        jax.ShapeDtypeStruct((NC, N_PAD, D), jnp.float32),
        jax.ShapeDtypeStruct((N_PAD, D), jnp.float32),
        jax.ShapeDtypeStruct((N_PAD,), jnp.float32),
    ),
    mesh=_mesh,
    scratch_types=[
        pltpu.VMEM((NCHUNK, K), jnp.int32),
        [pltpu.VMEM((K,), jnp.int32)] * NBUF,
        [pltpu.VMEM((K, D), jnp.float32)] * NBUF,
        pltpu.VMEM((K,), jnp.float32),
        pltpu.VMEM((ROWS_PER_TILE,), jnp.float32),
        pltpu.VMEM((ROWS_PER_TILE,), jnp.float32),
        pltpu.VMEM_SHARED((N_PAD, D), jnp.float32),
        pltpu.VMEM_SHARED((N_PAD,), jnp.float32),
        pltpu.SemaphoreType.DMA,
        pltpu.SemaphoreType.DMA,
        pltpu.SemaphoreType.DMA,
    ],
)
def _fused_kernel(x_hbm, col_hbm, row_hbm, acc_hbm, u_hbm, deg_hbm,
                  col_v, rowb, bufs, ones_v, zv, dinvv,
                  acc_sh, bins_sh, gsem, rsem, hsem):
    cid = lax.axis_index("c")
    sid = lax.axis_index("s")
    wid = sid * NC + cid
    r0 = sid * ROWS_PER_TILE
    tile_sl = pl.ds(r0, ROWS_PER_TILE)

    def zbody(i, _):
        r = i // (D // 16)
        c = lax.rem(i, D // 16)
        bufs[0][r, pl.ds(c * 16, 16)] = jnp.zeros((16,), jnp.float32)
        return 0

    lax.fori_loop(0, ZR * (D // 16), zbody, 0)

    def z1body(i, _):
        zv[pl.ds(i * 16, 16)] = jnp.zeros((16,), jnp.float32)
        return 0

    lax.fori_loop(0, ROWS_PER_TILE // 16, z1body, 0)
    for i in range(K // 16 + 1):
        o = min(i * 16, K - 16)
        ones_v[pl.ds(o, 16)] = jnp.ones((16,), jnp.float32)

    pltpu.sync_copy(zv, bins_sh.at[tile_sl])

    @pl.when(cid == 1)
    def _():
        zsrc = bufs[0].at[pl.ds(0, ZR), :]
        for i in range(ROWS_PER_TILE // ZR):
            pltpu.sync_copy(zsrc, acc_sh.at[pl.ds(r0 + i * ZR, ZR), :])

    plsc.subcore_barrier()

    for i in range(NBUF):
        pltpu.async_copy(
            x_hbm.at[pl.ds(r0 + i * ZR, ZR), :],
            bufs[i].at[pl.ds(0, ZR), :], gsem)

    GRP = 16
    for w01 in range(2):
        pltpu.sync_copy(row_hbm.at[sid * 2 + w01], col_v)

        def hbody(j, _):
            descs = []
            for b in range(GRP):
                descs.append(pltpu.async_copy(
                    ones_v, bins_sh.at[col_v.at[j * GRP + b]], hsem,
                    add=True))
            for dsc in descs:
                dsc.wait()
            return 0

        lax.fori_loop(0, NCHUNK // GRP, hbody, 0)
    plsc.subcore_barrier()

    pltpu.sync_copy(bins_sh.at[tile_sl], zv)

    @pl.when(cid == 0)
    def _():
        pltpu.async_copy(bins_sh.at[tile_sl], deg_hbm.at[tile_sl], hsem)

    def dbody(i, _):
        sl = pl.ds(i * 16, 16)
        dinvv[sl] = _vec_rsqrt(zv[sl] + 1.0)
        return 0

    lax.fori_loop(0, ROWS_PER_TILE // 16, dbody, 0)

    NUC = ROWS_PER_TILE // ZR
    nuc = jnp.where(r0 + ROWS_PER_TILE > N_NODES, (N_NODES - r0) // ZR, NUC)
    pltpu.async_copy(col_hbm.at[wid], col_v, rsem)

    def ubody(i2, _):
        for b in range(NBUF):
            i = i2 * NBUF + b

            @pl.when(i < nuc)
            def _():
                xsl = pl.ds(r0 + i * ZR, ZR)
                pltpu.make_async_copy(
                    x_hbm.at[xsl, :], bufs[b].at[pl.ds(0, ZR), :], gsem).wait()

                def sgroup(gidx, _):
                    dv = dinvv[pl.ds(i * ZR + gidx * 16, 16)]
                    for r in range(16):
                        s = dv[r]
                        row = gidx * 16 + r
                        for c in range(D // 16):
                            lsl = pl.ds(c * 16, 16)
                            bufs[b][row, lsl] = bufs[b][row, lsl] * s
                    return 0

                lax.fori_loop(0, ZR // 16, sgroup, 0)
                pltpu.sync_copy(bufs[b].at[pl.ds(0, ZR), :], u_hbm.at[xsl, :])

                @pl.when(cid == 0)
                def _():
                    pltpu.sync_copy(
                        bufs[b].at[pl.ds(0, ZR), :], acc_sh.at[xsl, :])

                @pl.when(i + NBUF < nuc)
                def _():
                    pltpu.async_copy(
                        x_hbm.at[pl.ds(r0 + (i + NBUF) * ZR, ZR), :],
                        bufs[b].at[pl.ds(0, ZR), :], gsem)

        return 0

    lax.fori_loop(0, NUC // NBUF, ubody, 0)

    pltpu.make_async_copy(col_hbm.at[wid], col_v, rsem).wait()

    @pl.when(cid == 0)
    def _():
        pltpu.make_async_copy(
            bins_sh.at[tile_sl], deg_hbm.at[tile_sl], hsem).wait()

    plsc.subcore_barrier()

    for b in range(NBUF):
        pltpu.async_copy(row_hbm.at[wid, b], rowb[b], rsem)
        pltpu.async_copy(u_hbm.at[col_v.at[b]], bufs[b], gsem)

    def body(j, _):
        for b in range(NBUF):
            jj = j * NBUF + b
            pltpu.make_async_copy(row_hbm.at[wid, jj], rowb[b], rsem).wait()
            pltpu.make_async_copy(u_hbm.at[col_v.at[jj]], bufs[b], gsem).wait()
            pltpu.sync_copy(bufs[b], acc_sh.at[rowb[b]], add=True)

            @pl.when(jj + NBUF < NCHUNK)
            def _():
                pltpu.async_copy(row_hbm.at[wid, jj + NBUF], rowb[b], rsem)
                pltpu.async_copy(u_hbm.at[col_v.at[jj + NBUF]], bufs[b], gsem)

        return 0

    lax.fori_loop(0, NCHUNK // NBUF, body, 0)
    plsc.subcore_barrier()
    pltpu.sync_copy(acc_sh.at[tile_sl, :], acc_hbm.at[cid, tile_sl, :])


BR = 1000


@functools.partial(
    pl.pallas_call,
    out_shape=jax.ShapeDtypeStruct((N_NODES, D), jnp.float32),
    grid=(N_NODES // BR,),
    in_specs=[
        pl.BlockSpec((NC, BR, D), lambda i: (0, i, 0)),
        pl.BlockSpec((D, D), lambda i: (0, 0)),
        pl.BlockSpec((BR, 1), lambda i: (i, 0)),
        pl.BlockSpec((1, D), lambda i: (0, 0)),
    ],
    out_specs=pl.BlockSpec((BR, D), lambda i: (i, 0)),
)
def _final_matmul(acc_ref, w_ref, deg_ref, b_ref, out_ref):
    dinv = lax.rsqrt(deg_ref[...] + 1.0)
    s = acc_ref[0] + acc_ref[1]
    y = jnp.dot(s.astype(jnp.bfloat16), w_ref[...].astype(jnp.bfloat16),
                preferred_element_type=jnp.float32)
    out_ref[...] = dinv * y + b_ref[...]


def kernel(x, edge_index, W, b):
    row = edge_index[0].reshape(NW, NCHUNK, K)
    col = edge_index[1].reshape(NW, NCHUNK, K)
    acc, _u, deg = _fused_kernel(x, col, row)
    out = _final_matmul(acc, W, deg.reshape(N_PAD, 1), b.reshape(1, D))
    return out
